# Initial kernel scaffold; baseline (speedup 1.0000x reference)
import jax
import jax.numpy as jnp
from jax.experimental import pallas as pl


def kernel(x, edge_index, edge_weight, params):
    raise NotImplementedError("write your pallas kernel here")


# speedup vs baseline: 4.2839x; 4.2839x over previous
import functools

import jax
import jax.numpy as jnp
from jax import lax
from jax.experimental import pallas as pl
from jax.experimental.pallas import tpu as pltpu
from jax.experimental.pallas import tpu_sc as plsc

NC = 2
NS = 16
NW = NC * NS


def _lnorm(h, eps=1e-5):
    m = jnp.mean(h, axis=-1, keepdims=True)
    v = jnp.mean((h - m) ** 2, axis=-1, keepdims=True)
    return (h - m) * jax.lax.rsqrt(v + eps)


def _mlp_relu(h, w1, b1, w2, b2):
    h = _lnorm(h)
    h = jax.nn.relu(jnp.dot(h, w1, preferred_element_type=jnp.float32) + b1)
    h = _lnorm(h)
    h = jnp.dot(h, w2, preferred_element_type=jnp.float32) + b2
    return jax.nn.relu(h)


def _tc_first(x, w1, b1, w2, b2, *, rows):
    n, d = x.shape
    h = w1.shape[1]

    def body(x_ref, w1_ref, b1_ref, w2_ref, b2_ref, o_ref):
        o_ref[...] = _mlp_relu(x_ref[...], w1_ref[...], b1_ref[...],
                               w2_ref[...], b2_ref[...])

    full = lambda a: pl.BlockSpec(a.shape, lambda i: (0,) * a.ndim)
    return pl.pallas_call(
        body,
        grid=(n // rows,),
        in_specs=[pl.BlockSpec((rows, d), lambda i: (i, 0)),
                  full(w1), full(b1), full(w2), full(b2)],
        out_specs=pl.BlockSpec((rows, h), lambda i: (i, 0)),
        out_shape=jax.ShapeDtypeStruct((n, h), jnp.float32),
    )(x, w1, b1, w2, b2)


def _tc_mid(p, dw1, db1, dw2, db2, ew1, eb1, ew2, eb2, *, rows):
    _, n, h = p.shape

    def body(p_ref, dw1_ref, db1_ref, dw2_ref, db2_ref,
             ew1_ref, eb1_ref, ew2_ref, eb2_ref, o_ref):
        agg = p_ref[0] + p_ref[1]
        hmid = _mlp_relu(agg, dw1_ref[...], db1_ref[...],
                         dw2_ref[...], db2_ref[...])
        o_ref[...] = _mlp_relu(hmid, ew1_ref[...], eb1_ref[...],
                               ew2_ref[...], eb2_ref[...])

    full = lambda a: pl.BlockSpec(a.shape, lambda i: (0,) * a.ndim)
    ws = [dw1, db1, dw2, db2, ew1, eb1, ew2, eb2]
    return pl.pallas_call(
        body,
        grid=(n // rows,),
        in_specs=[pl.BlockSpec((2, rows, h), lambda i: (0, i, 0))]
                 + [full(a) for a in ws],
        out_specs=pl.BlockSpec((rows, h), lambda i: (i, 0)),
        out_shape=jax.ShapeDtypeStruct((n, h), jnp.float32),
    )(p, *ws)


def _tc_final(p, dw1, db1, dw2, db2, cw1, cb1, cw2, cb2, *, rows):
    _, n, h = p.shape
    c = cw2.shape[1]

    def body(p_ref, dw1_ref, db1_ref, dw2_ref, db2_ref,
             cw1_ref, cb1_ref, cw2_ref, cb2_ref, o_ref):
        agg = p_ref[0] + p_ref[1]
        hm = _mlp_relu(agg, dw1_ref[...], db1_ref[...],
                       dw2_ref[...], db2_ref[...])
        hm = jax.nn.relu(
            jnp.dot(hm, cw1_ref[...], preferred_element_type=jnp.float32)
            + cb1_ref[...])
        hm = _lnorm(hm)
        o_ref[...] = (jnp.dot(hm, cw2_ref[...],
                              preferred_element_type=jnp.float32)
                      + cb2_ref[...])

    full = lambda a: pl.BlockSpec(a.shape, lambda i: (0,) * a.ndim)
    ws = [dw1, db1, dw2, db2, cw1, cb1, cw2, cb2]
    return pl.pallas_call(
        body,
        grid=(n // rows,),
        in_specs=[pl.BlockSpec((2, rows, h), lambda i: (0, i, 0))]
                 + [full(a) for a in ws],
        out_specs=pl.BlockSpec((rows, c), lambda i: (i, 0)),
        out_shape=jax.ShapeDtypeStruct((n, c), jnp.float32),
    )(p, *ws)


@functools.partial(jax.jit, static_argnames=("b",))
def _sc_scatter(h, gidx3, sidx3, w2, zeros, *, b):
    n, hdim = h.shape
    nb = gidx3.shape[1]
    rs = (n // NS) // 8 * 8
    tail = n - NS * rs

    mesh = plsc.VectorSubcoreMesh(core_axis_name="c", subcore_axis_name="s")

    @functools.partial(
        pl.kernel,
        mesh=mesh,
        compiler_params=pltpu.CompilerParams(use_tc_tiling_on_sc=False),
        out_type=jax.ShapeDtypeStruct((NC, n, hdim), jnp.float32),
        scratch_types=[
            pltpu.VMEM((nb, b), jnp.int32),
            pltpu.VMEM((nb, b), jnp.int32),
            pltpu.VMEM((nb * b,), jnp.float32),
            pltpu.VMEM((b, hdim), jnp.float32),
            pltpu.VMEM_SHARED((n, hdim), jnp.float32),
            pltpu.SemaphoreType.DMA,
        ],
    )
    def scat(h_hbm, g_hbm, s_hbm, w_hbm, z_hbm, out_hbm,
             g_v, s_v, w_v, rows_v, acc_sh, sem):
        cid = lax.axis_index("c")
        sid = lax.axis_index("s")
        wid = sid * NC + cid

        pltpu.sync_copy(g_hbm.at[wid], g_v)
        pltpu.sync_copy(s_hbm.at[wid], s_v)
        pltpu.sync_copy(w_hbm.at[wid], w_v)

        pltpu.sync_copy(z_hbm.at[pl.ds(sid * rs, rs)],
                        acc_sh.at[pl.ds(sid * rs, rs)])
        if tail:
            @pl.when(sid == NS - 1)
            def _():
                pltpu.sync_copy(z_hbm.at[pl.ds(NS * rs, tail)],
                                acc_sh.at[pl.ds(NS * rs, tail)])
        plsc.subcore_barrier()

        @pl.loop(0, nb)
        def _batch(i):
            pltpu.async_copy(h_hbm.at[g_v.at[i]], rows_v, sem).wait()

            @pl.loop(0, b // 16)
            def _grp(g):
                wch = w_v[pl.ds(i * b + g * 16, 16)]
                for j in range(16):
                    w16 = jnp.full((16,), wch[j])
                    e = g * 16 + j
                    for cchunk in range(hdim // 16):
                        sl = pl.ds(cchunk * 16, 16)
                        rows_v[e, sl] = rows_v[e, sl] * w16

            pltpu.sync_copy(rows_v, acc_sh.at[s_v.at[i]], add=True)

        plsc.subcore_barrier()
        pltpu.sync_copy(acc_sh.at[pl.ds(sid * rs, rs)],
                        out_hbm.at[cid, pl.ds(sid * rs, rs)])
        if tail:
            @pl.when(sid == NS - 1)
            def _():
                pltpu.sync_copy(acc_sh.at[pl.ds(NS * rs, tail)],
                                out_hbm.at[cid, pl.ds(NS * rs, tail)])

    return scat(h, gidx3, sidx3, w2, zeros)


def kernel(x, edge_index, edge_weight, params):
    n, d = x.shape
    eg = edge_weight.shape[0]
    p = params

    epw = eg // NW
    b = 80
    assert eg % NW == 0 and epw % b == 0 and n % NS == 0
    nb = epw // b

    src = edge_index[0].reshape(NW, nb, b)
    dst = edge_index[1].reshape(NW, nb, b)
    w2 = edge_weight.reshape(NW, nb * b)
    zeros = jnp.zeros((n, p['v2e0_eW2'].shape[1]), jnp.float32)

    r1 = lambda v: v.reshape(1, -1)
    rows = 1000

    def wset(name):
        return (p[name + '_eW1'], r1(p[name + '_eb1']),
                p[name + '_eW2'], r1(p[name + '_eb2']),
                p[name + '_dW1'], r1(p[name + '_db1']),
                p[name + '_dW2'], r1(p[name + '_db2']))

    v2e0 = wset('v2e0'); e2v0 = wset('e2v0')
    v2e1 = wset('v2e1'); e2v1 = wset('e2v1')

    h = _tc_first(x, *v2e0[:4], rows=rows)
    pp = _sc_scatter(h, src, dst, w2, zeros, b=b)
    h = _tc_mid(pp, *v2e0[4:], *e2v0[:4], rows=rows)
    pp = _sc_scatter(h, dst, src, w2, zeros, b=b)
    h = _tc_mid(pp, *e2v0[4:], *v2e1[:4], rows=rows)
    pp = _sc_scatter(h, src, dst, w2, zeros, b=b)
    h = _tc_mid(pp, *v2e1[4:], *e2v1[:4], rows=rows)
    pp = _sc_scatter(h, dst, src, w2, zeros, b=b)
    out = _tc_final(pp, *e2v1[4:],
                    p['clf_W1'], r1(p['clf_b1']),
                    p['clf_W2'], r1(p['clf_b2']), rows=rows)
    return out


# speedup vs baseline: 5.7341x; 1.3385x over previous
import functools

import jax
import jax.numpy as jnp
from jax import lax
from jax.experimental import pallas as pl
from jax.experimental.pallas import tpu as pltpu
from jax.experimental.pallas import tpu_sc as plsc

NC = 2
NS = 16
NW = NC * NS


def _lnorm(h, eps=1e-5):
    m = jnp.mean(h, axis=-1, keepdims=True)
    v = jnp.mean((h - m) ** 2, axis=-1, keepdims=True)
    return (h - m) * jax.lax.rsqrt(v + eps)


def _mlp_relu(h, w1, b1, w2, b2):
    h = _lnorm(h)
    h = jax.nn.relu(jnp.dot(h, w1, preferred_element_type=jnp.float32) + b1)
    h = _lnorm(h)
    h = jnp.dot(h, w2, preferred_element_type=jnp.float32) + b2
    return jax.nn.relu(h)


def _tc_first(x, w1, b1, w2, b2, *, rows):
    n, d = x.shape
    h = w1.shape[1]

    def body(x_ref, w1_ref, b1_ref, w2_ref, b2_ref, o_ref):
        o_ref[...] = _mlp_relu(x_ref[...], w1_ref[...], b1_ref[...],
                               w2_ref[...], b2_ref[...])

    full = lambda a: pl.BlockSpec(a.shape, lambda i: (0,) * a.ndim)
    return pl.pallas_call(
        body,
        grid=(n // rows,),
        in_specs=[pl.BlockSpec((rows, d), lambda i: (i, 0)),
                  full(w1), full(b1), full(w2), full(b2)],
        out_specs=pl.BlockSpec((rows, h), lambda i: (i, 0)),
        out_shape=jax.ShapeDtypeStruct((n, h), jnp.float32),
    )(x, w1, b1, w2, b2)


def _tc_mid(p, dw1, db1, dw2, db2, ew1, eb1, ew2, eb2, *, rows):
    _, n, h = p.shape

    def body(p_ref, dw1_ref, db1_ref, dw2_ref, db2_ref,
             ew1_ref, eb1_ref, ew2_ref, eb2_ref, o_ref):
        agg = p_ref[0] + p_ref[1]
        hmid = _mlp_relu(agg, dw1_ref[...], db1_ref[...],
                         dw2_ref[...], db2_ref[...])
        o_ref[...] = _mlp_relu(hmid, ew1_ref[...], eb1_ref[...],
                               ew2_ref[...], eb2_ref[...])

    full = lambda a: pl.BlockSpec(a.shape, lambda i: (0,) * a.ndim)
    ws = [dw1, db1, dw2, db2, ew1, eb1, ew2, eb2]
    return pl.pallas_call(
        body,
        grid=(n // rows,),
        in_specs=[pl.BlockSpec((2, rows, h), lambda i: (0, i, 0))]
                 + [full(a) for a in ws],
        out_specs=pl.BlockSpec((rows, h), lambda i: (i, 0)),
        out_shape=jax.ShapeDtypeStruct((n, h), jnp.float32),
    )(p, *ws)


def _tc_final(p, dw1, db1, dw2, db2, cw1, cb1, cw2, cb2, *, rows):
    _, n, h = p.shape
    c = cw2.shape[1]

    def body(p_ref, dw1_ref, db1_ref, dw2_ref, db2_ref,
             cw1_ref, cb1_ref, cw2_ref, cb2_ref, o_ref):
        agg = p_ref[0] + p_ref[1]
        hm = _mlp_relu(agg, dw1_ref[...], db1_ref[...],
                       dw2_ref[...], db2_ref[...])
        hm = jax.nn.relu(
            jnp.dot(hm, cw1_ref[...], preferred_element_type=jnp.float32)
            + cb1_ref[...])
        hm = _lnorm(hm)
        o_ref[...] = (jnp.dot(hm, cw2_ref[...],
                              preferred_element_type=jnp.float32)
                      + cb2_ref[...])

    full = lambda a: pl.BlockSpec(a.shape, lambda i: (0,) * a.ndim)
    ws = [dw1, db1, dw2, db2, cw1, cb1, cw2, cb2]
    return pl.pallas_call(
        body,
        grid=(n // rows,),
        in_specs=[pl.BlockSpec((2, rows, h), lambda i: (0, i, 0))]
                 + [full(a) for a in ws],
        out_specs=pl.BlockSpec((rows, c), lambda i: (i, 0)),
        out_shape=jax.ShapeDtypeStruct((n, c), jnp.float32),
    )(p, *ws)


@functools.partial(jax.jit, static_argnames=("b",))
def _sc_scatter(h, gidx3, sidx3, w2, zeros, *, b):
    n, hdim = h.shape
    nb = gidx3.shape[1]
    nch = hdim // 16
    assert nb % 4 == 0 and b % 16 == 0
    rs = (n // NS) // 8 * 8
    tail = n - NS * rs

    mesh = plsc.VectorSubcoreMesh(core_axis_name="c", subcore_axis_name="s")

    @functools.partial(
        pl.kernel,
        mesh=mesh,
        compiler_params=pltpu.CompilerParams(use_tc_tiling_on_sc=False),
        out_type=jax.ShapeDtypeStruct((NC, n, hdim), jnp.float32),
        scratch_types=[
            pltpu.VMEM((nb, b), jnp.int32),
            pltpu.VMEM((nb, b), jnp.int32),
            pltpu.VMEM((nb * b,), jnp.float32),
            pltpu.VMEM((b, hdim), jnp.float32),
            pltpu.VMEM((b, hdim), jnp.float32),
            pltpu.VMEM((b, hdim), jnp.float32),
            pltpu.VMEM((b, hdim), jnp.float32),
            pltpu.VMEM_SHARED((n, hdim), jnp.float32),
            pltpu.SemaphoreType.DMA,
            pltpu.SemaphoreType.DMA,
            pltpu.SemaphoreType.DMA,
            pltpu.SemaphoreType.DMA,
            pltpu.SemaphoreType.DMA,
            pltpu.SemaphoreType.DMA,
            pltpu.SemaphoreType.DMA,
            pltpu.SemaphoreType.DMA,
        ],
    )
    def scat(h_hbm, g_hbm, s_hbm, w_hbm, z_hbm, out_hbm,
             g_v, s_v, w_v, r0, r1, r2, r3, acc_sh,
             sg0, sg1, sg2, sg3, ss0, ss1, ss2, ss3):
        cid = lax.axis_index("c")
        sid = lax.axis_index("s")
        wid = sid * NC + cid
        bufs = [r0, r1, r2, r3]
        sg = [sg0, sg1, sg2, sg3]
        ss = [ss0, ss1, ss2, ss3]

        pltpu.sync_copy(g_hbm.at[wid], g_v)
        pltpu.sync_copy(s_hbm.at[wid], s_v)
        pltpu.sync_copy(w_hbm.at[wid], w_v)

        pltpu.sync_copy(z_hbm.at[pl.ds(sid * rs, rs)],
                        acc_sh.at[pl.ds(sid * rs, rs)])
        if tail:
            @pl.when(sid == NS - 1)
            def _():
                pltpu.sync_copy(z_hbm.at[pl.ds(NS * rs, tail)],
                                acc_sh.at[pl.ds(NS * rs, tail)])
        plsc.subcore_barrier()

        def start_gather(i, k):
            pltpu.async_copy(h_hbm.at[g_v.at[i]], bufs[k], sg[k])

        def wait_gather(k):
            pltpu.make_async_copy(h_hbm.at[g_v.at[0]], bufs[k], sg[k]).wait()

        def start_scatter(i, k):
            pltpu.async_copy(bufs[k], acc_sh.at[s_v.at[i]], ss[k], add=True)

        def wait_scatter(k):
            pltpu.make_async_copy(bufs[k], acc_sh.at[s_v.at[0]],
                                  ss[k]).wait()

        def scale(i, k):
            rows = bufs[k]

            @plsc.parallel_loop(0, b // 16)
            def _grp(g):
                wch = w_v[pl.ds(i * b + g * 16, 16)]
                for jb in range(4):
                    es = g * 16 + jb * 4
                    prods = []
                    for j in range(4):
                        w16 = jnp.full((16,), wch[jb * 4 + j])
                        for c in range(nch):
                            prods.append(
                                rows[es + j, pl.ds(c * 16, 16)] * w16)
                    t = 0
                    for j in range(4):
                        for c in range(nch):
                            rows[es + j, pl.ds(c * 16, 16)] = prods[t]
                            t += 1

        start_gather(0, 0)
        start_gather(1, 1)
        for k in range(4):
            if k >= 2:
                wait_scatter((k + 2) % 4)
            start_gather(k + 2, (k + 2) % 4)
            wait_gather(k)
            scale(k, k)
            start_scatter(k, k)

        @pl.loop(1, nb // 4 - 1)
        def _outer(o):
            i0 = o * 4
            for k in range(4):
                wait_scatter((k + 2) % 4)
                start_gather(i0 + k + 2, (k + 2) % 4)
                wait_gather(k)
                scale(i0 + k, k)
                start_scatter(i0 + k, k)

        i0 = nb - 4
        for k in range(4):
            wait_scatter((k + 2) % 4)
            if k < 2:
                start_gather(i0 + k + 2, (k + 2) % 4)
            wait_gather(k)
            scale(i0 + k, k)
            start_scatter(i0 + k, k)
        wait_scatter(2)
        wait_scatter(3)

        plsc.subcore_barrier()
        pltpu.sync_copy(acc_sh.at[pl.ds(sid * rs, rs)],
                        out_hbm.at[cid, pl.ds(sid * rs, rs)])
        if tail:
            @pl.when(sid == NS - 1)
            def _():
                pltpu.sync_copy(acc_sh.at[pl.ds(NS * rs, tail)],
                                out_hbm.at[cid, pl.ds(NS * rs, tail)])

    return scat(h, gidx3, sidx3, w2, zeros)


def kernel(x, edge_index, edge_weight, params):
    n, d = x.shape
    eg = edge_weight.shape[0]
    p = params

    b = 128
    epw = -(-eg // (NW * 4 * b)) * 4 * b
    pad = NW * epw - eg
    nb = epw // b
    assert n % NS == 0

    def padded(a):
        return jnp.concatenate([a, jnp.zeros((pad,), a.dtype)])

    src = padded(edge_index[0]).reshape(NW, nb, b)
    dst = padded(edge_index[1]).reshape(NW, nb, b)
    w2 = padded(edge_weight).reshape(NW, nb * b)
    zeros = jnp.zeros((n, p['v2e0_eW2'].shape[1]), jnp.float32)

    r1 = lambda v: v.reshape(1, -1)
    rows = 1000

    def wset(name):
        return (p[name + '_eW1'], r1(p[name + '_eb1']),
                p[name + '_eW2'], r1(p[name + '_eb2']),
                p[name + '_dW1'], r1(p[name + '_db1']),
                p[name + '_dW2'], r1(p[name + '_db2']))

    v2e0 = wset('v2e0'); e2v0 = wset('e2v0')
    v2e1 = wset('v2e1'); e2v1 = wset('e2v1')

    h = _tc_first(x, *v2e0[:4], rows=rows)
    pp = _sc_scatter(h, src, dst, w2, zeros, b=b)
    h = _tc_mid(pp, *v2e0[4:], *e2v0[:4], rows=rows)
    pp = _sc_scatter(h, dst, src, w2, zeros, b=b)
    h = _tc_mid(pp, *e2v0[4:], *v2e1[:4], rows=rows)
    pp = _sc_scatter(h, src, dst, w2, zeros, b=b)
    h = _tc_mid(pp, *v2e1[4:], *e2v1[:4], rows=rows)
    pp = _sc_scatter(h, dst, src, w2, zeros, b=b)
    out = _tc_final(pp, *e2v1[4:],
                    p['clf_W1'], r1(p['clf_b1']),
                    p['clf_W2'], r1(p['clf_b2']), rows=rows)
    return out
